# baseline (speedup 1.0000x reference)
import functools
import jax
import jax.numpy as jnp
from jax import lax
from jax.experimental import pallas as pl
from jax.experimental.pallas import tpu as pltpu
from jax.experimental.pallas import tpu_sc as plsc

NW = 32
CHUNK_E = 32768
CHUNK_W = CHUNK_E // 4


def _sc_add_by_mask(total_e):
    per_w = total_e // NW
    n_chunks = per_w // CHUNK_E
    mesh = plsc.VectorSubcoreMesh(core_axis_name="c", subcore_axis_name="s")

    @functools.partial(
        pl.kernel,
        mesh=mesh,
        out_type=jax.ShapeDtypeStruct((total_e,), jnp.float32),
        compiler_params=pltpu.CompilerParams(needs_layout_passes=False),
        scratch_types=[
            pltpu.VMEM((CHUNK_E,), jnp.float32),
            pltpu.VMEM((CHUNK_W,), jnp.uint32),
        ],
    )
    def k(x_hbm, m_hbm, out_hbm, xbuf, mbuf):
        wid = lax.axis_index("s") * 2 + lax.axis_index("c")
        base = wid * per_w

        iota4 = lax.iota(jnp.int32, 16) * 4

        def chunk_body(ci, _):
            off = pl.multiple_of(base + ci * CHUNK_E, 8)
            off_w = pl.multiple_of(base // 4 + ci * CHUNK_W, 8)
            pltpu.sync_copy(x_hbm.at[pl.ds(off, CHUNK_E)], xbuf)
            pltpu.sync_copy(m_hbm.at[pl.ds(off_w, CHUNK_W)], mbuf)

            def grp_body(g, _):
                w = mbuf[pl.ds(g * 16, 16)]
                for j in range(4):
                    mj = ((w >> (8 * j)) & jnp.uint32(1)).astype(jnp.float32)
                    idx = iota4 + (g * 64 + j)
                    xj = plsc.load_gather(xbuf, [idx])
                    plsc.store_scatter(xbuf, [idx], xj + mj)
                return 0

            lax.fori_loop(0, CHUNK_E // 64, grp_body, 0)
            pltpu.sync_copy(xbuf, out_hbm.at[pl.ds(off, CHUNK_E)])
            return 0

        lax.fori_loop(0, n_chunks, chunk_body, 0)

    return k


def kernel(x, mask):
    R, C = x.shape
    total = R * C
    m8 = mask.view(jnp.int8)
    m32 = lax.bitcast_convert_type(m8.reshape(R, C // 4, 4), jnp.uint32)
    out = _sc_add_by_mask(total)(x.reshape(total), m32.reshape(total // 4))
    return out.reshape(R, C)


# speedup vs baseline: 5.9618x; 5.9618x over previous
import jax
import jax.numpy as jnp
from jax import lax
from jax.experimental import pallas as pl
from jax.experimental.pallas import tpu as pltpu

BR = 2048


def _body(x_ref, b_ref, o_ref):
    bi = b_ref[...].astype(jnp.int32)
    bexp = jnp.repeat(bi, 8, axis=0)
    sh = lax.broadcasted_iota(jnp.int32, bexp.shape, 0) % 8
    m = ((bexp >> sh) & 1).astype(jnp.float32)
    o_ref[...] = x_ref[...] + m


def kernel(x, mask):
    R, C = x.shape
    w = (jnp.uint8(1) << jnp.arange(8, dtype=jnp.uint8))
    bits = jnp.sum(
        mask.reshape(R // 8, 8, C) * w[None, :, None], axis=1, dtype=jnp.uint8
    )
    return pl.pallas_call(
        _body,
        grid=(R // BR,),
        in_specs=[
            pl.BlockSpec((BR, C), lambda i: (i, 0)),
            pl.BlockSpec((BR // 8, C), lambda i: (i, 0)),
        ],
        out_specs=pl.BlockSpec((BR, C), lambda i: (i, 0)),
        out_shape=jax.ShapeDtypeStruct((R, C), x.dtype),
    )(x, bits)
